# baseline (speedup 1.0000x reference)
import functools

import jax
import jax.numpy as jnp
from jax import lax
from jax.experimental import pallas as pl
from jax.experimental.pallas import tpu as pltpu
from jax.experimental.pallas import tpu_sc as plsc

_BATCH = 4
_NUM_CORES = 2
_NUM_SUBCORES = 16
_NUM_WORKERS = _NUM_CORES * _NUM_SUBCORES
_CHUNK = 64
_SP_CHUNK = 32


def kernel(x, pe_weight):
    batch = x.shape[0]
    max_len, d_model = pe_weight.shape
    assert batch == _BATCH
    rows_per_worker = max_len // _NUM_WORKERS
    assert rows_per_worker == 3 * _CHUNK + 2 * _SP_CHUNK

    mesh = plsc.VectorSubcoreMesh(core_axis_name="c", subcore_axis_name="s")

    @functools.partial(
        pl.kernel,
        mesh=mesh,
        out_type=jax.ShapeDtypeStruct((batch, max_len, d_model), pe_weight.dtype),
        scratch_types=(
            [pltpu.VMEM((_CHUNK, d_model), pe_weight.dtype)] * 2
            + [pltpu.VMEM_SHARED((_NUM_SUBCORES * _SP_CHUNK, d_model), pe_weight.dtype)]
            + [pltpu.SemaphoreType.DMA] * 3
            + [pltpu.SemaphoreType.DMA] * 3
        ),
    )
    def _sc_bcast(pe_hbm, out_hbm, tile0, tile1, shared, *sems):
        in_sems = sems[:3]
        out_sems = sems[3:]
        sid = lax.axis_index("s")
        wid = sid * _NUM_CORES + lax.axis_index("c")
        base = wid * rows_per_worker
        sp = shared.at[pl.ds(sid * _SP_CHUNK, _SP_CHUNK)]
        stages = [tile0, tile1, sp]
        chunks = [
            (0, 0, _CHUNK),
            (1, _CHUNK, _CHUNK),
            (2, 2 * _CHUNK, _SP_CHUNK),
            (2, 2 * _CHUNK + _SP_CHUNK, _SP_CHUNK),
            (0, 3 * _CHUNK, _CHUNK),
        ]

        def in_copy(ci):
            slot, off, rows = chunks[ci]
            return pltpu.make_async_copy(
                pe_hbm.at[pl.ds(base + off, rows)], stages[slot], in_sems[slot]
            )

        def out_copies(ci):
            slot, off, rows = chunks[ci]
            return [
                pltpu.make_async_copy(
                    stages[slot],
                    out_hbm.at[b, pl.ds(base + off, rows)],
                    out_sems[slot],
                )
                for b in range(_BATCH)
            ]

        for ci in range(3):
            in_copy(ci).start()
        for ci in range(3):
            in_copy(ci).wait()
            for c in out_copies(ci):
                c.start()
        for c in out_copies(2):
            c.wait()
        in_copy(3).start()
        for c in out_copies(0):
            c.wait()
        in_copy(4).start()
        in_copy(3).wait()
        for c in out_copies(3):
            c.start()
        in_copy(4).wait()
        for c in out_copies(4):
            c.start()
        for ci in (1, 3, 4):
            for c in out_copies(ci):
                c.wait()

    return _sc_bcast(pe_weight)


# speedup vs baseline: 1.0198x; 1.0198x over previous
import functools

import jax
import jax.numpy as jnp
from jax import lax
from jax.experimental import pallas as pl
from jax.experimental.pallas import tpu as pltpu
from jax.experimental.pallas import tpu_sc as plsc

_BATCH = 4
_NUM_CORES = 2
_NUM_SUBCORES = 16
_NUM_WORKERS = _NUM_CORES * _NUM_SUBCORES
_CHUNK = 128


def kernel(x, pe_weight):
    batch = x.shape[0]
    max_len, d_model = pe_weight.shape
    assert batch == _BATCH and max_len % _NUM_WORKERS == 0
    rows_per_worker = max_len // _NUM_WORKERS
    assert rows_per_worker % _CHUNK == 0
    n_chunks = rows_per_worker // _CHUNK

    mesh = plsc.VectorSubcoreMesh(core_axis_name="c", subcore_axis_name="s")

    @functools.partial(
        pl.kernel,
        mesh=mesh,
        out_type=jax.ShapeDtypeStruct((batch, max_len, d_model), pe_weight.dtype),
        scratch_types=[
            pltpu.VMEM((_CHUNK, d_model), pe_weight.dtype),
            pltpu.SemaphoreType.DMA,
        ],
    )
    def _sc_bcast(pe_hbm, out_hbm, buf, sem):
        wid = lax.axis_index("s") * _NUM_CORES + lax.axis_index("c")
        base = wid * rows_per_worker

        def body(i, _):
            row = base + i * _CHUNK
            pltpu.sync_copy(pe_hbm.at[pl.ds(row, _CHUNK)], buf)
            copies = [
                pltpu.make_async_copy(
                    buf, out_hbm.at[b, pl.ds(row, _CHUNK)], sem
                )
                for b in range(_BATCH)
            ]
            for c in copies:
                c.start()
            for c in copies:
                c.wait()
            return ()

        lax.fori_loop(0, n_chunks, body, ())

    return _sc_bcast(pe_weight)
